# baseline (speedup 1.0000x reference)
import functools

import jax
import jax.numpy as jnp
from jax import lax
from jax.experimental import pallas as pl
from jax.experimental.pallas import tpu as pltpu
from jax.experimental.pallas import tpu_sc as plsc

_V = 1_000_000
_L = 16384
_D = 64
_LANES = 16
_NC = 2
_NS = 16
_BPW = _L // (_NC * _NS)
_BLK = 98_304


def _sc_counts(tokens_4d, zeros_hbm):
    mesh = plsc.VectorSubcoreMesh(core_axis_name="c", subcore_axis_name="s")

    @functools.partial(
        pl.kernel,
        out_type=[jax.ShapeDtypeStruct((_V,), jnp.float32),
                  jax.ShapeDtypeStruct((_V,), jnp.float32)],
        mesh=mesh,
        scratch_types=[
            pltpu.VMEM((1, _BPW), jnp.int32),
            pltpu.VMEM((1, _BPW), jnp.float32),
            pltpu.VMEM_SHARED((_V,), jnp.float32),
        ],
    )
    def sc_kernel(tok_hbm, zeros_hbm_ref, out0, out1, idx_v, ones_v,
                  counts_sp):
        c = lax.axis_index("c")
        s = lax.axis_index("s")

        @pl.when(s == 0)
        def _():
            pltpu.sync_copy(zeros_hbm_ref, counts_sp)

        for i in range(0, _BPW, _LANES):
            ones_v[0, pl.ds(i, _LANES)] = jnp.ones((_LANES,), jnp.float32)
        pltpu.sync_copy(tok_hbm.at[c, s], idx_v)

        plsc.subcore_barrier()
        pltpu.sync_copy(ones_v.at[0], counts_sp.at[idx_v.at[0]], add=True)
        plsc.subcore_barrier()

        @pl.when(jnp.logical_and(s == 0, c == 0))
        def _():
            pltpu.sync_copy(counts_sp, out0)

        @pl.when(jnp.logical_and(s == 0, c == 1))
        def _():
            pltpu.sync_copy(counts_sp, out1)

    return sc_kernel(tokens_4d, zeros_hbm)


def _tc_logits(table_t, counts0, counts1, W, b_col):
    def body(p_ref, c0_ref, c1_ref, w_ref, b_ref, o_ref, acc_ref):
        i = pl.program_id(0)

        @pl.when(i == 0)
        def _():
            acc_ref[...] = jnp.zeros_like(acc_ref)

        cb = (c0_ref[...] + c1_ref[...])[None, :]
        last = pl.num_programs(0) - 1

        @pl.when(i < last)
        def _():
            prod = p_ref[...] * cb
            acc_ref[...] += jnp.sum(prod, axis=1, keepdims=True)

        @pl.when(i == last)
        def _():
            lane = jax.lax.broadcasted_iota(jnp.int32, (1, _BLK), 1)
            valid = lane < (_V - i * _BLK)
            prod = jnp.where(valid, p_ref[...] * cb, 0.0)
            acc_ref[...] += jnp.sum(prod, axis=1, keepdims=True)

        @pl.when(i == last)
        def _():
            enc = acc_ref[...] * (1.0 / _L)
            logits = jnp.dot(
                w_ref[...], enc,
                preferred_element_type=jnp.float32) + b_ref[...]
            o_ref[...] = logits[:, 0]

    return pl.pallas_call(
        body,
        grid=(pl.cdiv(_V, _BLK),),
        in_specs=[
            pl.BlockSpec((_D, _BLK), lambda i: (0, i)),
            pl.BlockSpec((_BLK,), lambda i: (i,)),
            pl.BlockSpec((_BLK,), lambda i: (i,)),
            pl.BlockSpec((5, _D), lambda i: (0, 0)),
            pl.BlockSpec((5, 1), lambda i: (0, 0)),
        ],
        out_specs=pl.BlockSpec((5,), lambda i: (0,)),
        out_shape=jax.ShapeDtypeStruct((5,), jnp.float32),
        scratch_shapes=[pltpu.VMEM((_D, 1), jnp.float32)],
        compiler_params=pltpu.CompilerParams(
            vmem_limit_bytes=100 * 1024 * 1024),
    )(table_t, counts0, counts1, W, b_col)


@jax.jit
def kernel(tokens, emb_table, W, b):
    tokens_4d = tokens.astype(jnp.int32).reshape(_NC, _NS, 1, _BPW)
    counts0, counts1 = _sc_counts(tokens_4d, jnp.zeros((_V,), jnp.float32))
    return _tc_logits(emb_table.T, counts0, counts1, W, b.reshape(5, 1))


# speedup vs baseline: 1.0313x; 1.0313x over previous
import functools

import jax
import jax.numpy as jnp
from jax import lax
from jax.experimental import pallas as pl
from jax.experimental.pallas import tpu as pltpu
from jax.experimental.pallas import tpu_sc as plsc

_V = 1_000_000
_L = 16384
_D = 64
_LANES = 16
_NC = 2
_NS = 16
_BPW = _L // (_NC * _NS)
_BLK = 65_536


def _sc_counts(tokens_4d, zeros_hbm):
    mesh = plsc.VectorSubcoreMesh(core_axis_name="c", subcore_axis_name="s")

    @functools.partial(
        pl.kernel,
        out_type=[jax.ShapeDtypeStruct((_V,), jnp.float32),
                  jax.ShapeDtypeStruct((_V,), jnp.float32)],
        mesh=mesh,
        scratch_types=[
            pltpu.VMEM((1, _BPW), jnp.int32),
            pltpu.VMEM((1, _BPW), jnp.float32),
            pltpu.VMEM_SHARED((_V,), jnp.float32),
        ],
    )
    def sc_kernel(tok_hbm, zeros_hbm_ref, out0, out1, idx_v, ones_v,
                  counts_sp):
        c = lax.axis_index("c")
        s = lax.axis_index("s")

        @pl.when(s == 0)
        def _():
            pltpu.sync_copy(zeros_hbm_ref, counts_sp)

        for i in range(0, _BPW, _LANES):
            ones_v[0, pl.ds(i, _LANES)] = jnp.ones((_LANES,), jnp.float32)
        pltpu.sync_copy(tok_hbm.at[c, s], idx_v)

        plsc.subcore_barrier()
        pltpu.sync_copy(ones_v.at[0], counts_sp.at[idx_v.at[0]], add=True)
        plsc.subcore_barrier()

        @pl.when(jnp.logical_and(s == 0, c == 0))
        def _():
            pltpu.sync_copy(counts_sp, out0)

        @pl.when(jnp.logical_and(s == 0, c == 1))
        def _():
            pltpu.sync_copy(counts_sp, out1)

    return sc_kernel(tokens_4d, zeros_hbm)


def _tc_logits(table_t, counts0, counts1, W, b_col):
    def body(p_ref, c0_ref, c1_ref, w_ref, b_ref, o_ref, acc_ref):
        i = pl.program_id(0)

        @pl.when(i == 0)
        def _():
            acc_ref[...] = jnp.zeros_like(acc_ref)

        cb = (c0_ref[...] + c1_ref[...])[None, :]
        last = pl.num_programs(0) - 1

        @pl.when(i < last)
        def _():
            prod = p_ref[...] * cb
            acc_ref[...] += jnp.sum(prod, axis=1, keepdims=True)

        @pl.when(i == last)
        def _():
            lane = jax.lax.broadcasted_iota(jnp.int32, (1, _BLK), 1)
            valid = lane < (_V - i * _BLK)
            prod = jnp.where(valid, p_ref[...] * cb, 0.0)
            acc_ref[...] += jnp.sum(prod, axis=1, keepdims=True)

        @pl.when(i == last)
        def _():
            enc = acc_ref[...] * (1.0 / _L)
            logits = jnp.dot(
                w_ref[...], enc,
                preferred_element_type=jnp.float32) + b_ref[...]
            o_ref[...] = logits[:, 0]

    return pl.pallas_call(
        body,
        grid=(pl.cdiv(_V, _BLK),),
        in_specs=[
            pl.BlockSpec((_D, _BLK), lambda i: (0, i)),
            pl.BlockSpec((_BLK,), lambda i: (i,)),
            pl.BlockSpec((_BLK,), lambda i: (i,)),
            pl.BlockSpec((5, _D), lambda i: (0, 0)),
            pl.BlockSpec((5, 1), lambda i: (0, 0)),
        ],
        out_specs=pl.BlockSpec((5,), lambda i: (0,)),
        out_shape=jax.ShapeDtypeStruct((5,), jnp.float32),
        scratch_shapes=[pltpu.VMEM((_D, 1), jnp.float32)],
        compiler_params=pltpu.CompilerParams(
            vmem_limit_bytes=100 * 1024 * 1024),
    )(table_t, counts0, counts1, W, b_col)


@jax.jit
def kernel(tokens, emb_table, W, b):
    tokens_4d = tokens.astype(jnp.int32).reshape(_NC, _NS, 1, _BPW)
    counts0, counts1 = _sc_counts(tokens_4d, jnp.zeros((_V,), jnp.float32))
    return _tc_logits(emb_table.T, counts0, counts1, W, b.reshape(5, 1))
